# baseline (speedup 1.0000x reference)
import jax
import jax.numpy as jnp
from jax.experimental import pallas as pl
from jax.experimental.pallas import tpu as pltpu

_K = 24
_D = 12
_CS = 2


def _copy_body(x_ref, o_ref, buf, in_sem, out_sem):
    f, b, c_dim = x_ref.shape[0], x_ref.shape[1], x_ref.shape[2]
    cc = c_dim // _CS
    chunks = [(i, j, s) for j in range(b) for i in range(f) for s in range(_CS)]
    nc = len(chunks)

    def _pair(c):
        i, j, s = chunks[c]
        src = x_ref.at[i, j, pl.ds(s * cc, cc)]
        dst = o_ref.at[j, i, pl.ds(s * cc, cc)]
        return src, dst

    def start_in(c):
        src, _ = _pair(c)
        pltpu.make_async_copy(src, buf.at[c % _K], in_sem.at[c % _K]).start()

    def wait_in(c):
        src, _ = _pair(c)
        pltpu.make_async_copy(src, buf.at[c % _K], in_sem.at[c % _K]).wait()

    def start_out(c):
        _, dst = _pair(c)
        pltpu.make_async_copy(buf.at[c % _K], dst, out_sem.at[c % _K]).start()

    def wait_out(c):
        _, dst = _pair(c)
        pltpu.make_async_copy(buf.at[c % _K], dst, out_sem.at[c % _K]).wait()

    for c in range(min(_K, nc)):
        start_in(c)
    waited = [False] * nc
    for c in range(nc):
        wait_in(c)
        start_out(c)
        m = c + _K - _D
        if _K <= m < nc:
            wait_out(m - _K)
            waited[m - _K] = True
            start_in(m)
    for c in range(nc):
        if not waited[c]:
            wait_out(c)


def kernel(features):
    F, B, C, H, W = features.shape
    out = pl.pallas_call(
        _copy_body,
        in_specs=[pl.BlockSpec(memory_space=pltpu.MemorySpace.HBM)],
        out_specs=pl.BlockSpec(memory_space=pltpu.MemorySpace.HBM),
        out_shape=jax.ShapeDtypeStruct((B, F, C, H, W), features.dtype),
        scratch_shapes=[
            pltpu.VMEM((_K, C // _CS, H, W), features.dtype),
            pltpu.SemaphoreType.DMA((_K,)),
            pltpu.SemaphoreType.DMA((_K,)),
        ],
    )(features)
    return jnp.reshape(out, (B, F * C * H, W))


# speedup vs baseline: 1.1276x; 1.1276x over previous
import functools

import jax
import jax.numpy as jnp
from jax import lax
from jax.experimental import pallas as pl
from jax.experimental.pallas import tpu as pltpu
from jax.experimental.pallas import tpu_sc as plsc

_PARTS = 32
_NB = 4
_LAG = 2


def kernel(features):
    F, B, C, H, W = features.shape
    R = C * H
    rows = R // _PARTS
    x4 = jnp.reshape(features, (F, B, R, W))

    info = plsc.get_sparse_core_info()
    nw = info.num_cores * info.num_subcores
    n_chunks = F * B * _PARTS
    per_w = n_chunks // nw

    mesh = plsc.VectorSubcoreMesh(core_axis_name="c", subcore_axis_name="s")

    @functools.partial(
        pl.kernel,
        out_type=jax.ShapeDtypeStruct((B, F, R, W), features.dtype),
        mesh=mesh,
        scratch_types=[
            pltpu.VMEM((_NB, rows, W), features.dtype),
            pltpu.SemaphoreType.DMA((_NB,)),
            pltpu.SemaphoreType.DMA((_NB,)),
        ],
    )
    def sc_copy(x_hbm, o_hbm, buf, in_sem, out_sem):
        wid = lax.axis_index("s") * info.num_cores + lax.axis_index("c")

        def refs(t, s):
            g = wid * per_w + t
            i = g // (B * _PARTS)
            j = (g // _PARTS) % B
            p = g % _PARTS
            src = x_hbm.at[i, j, pl.ds(p * rows, rows)]
            dst = o_hbm.at[j, i, pl.ds(p * rows, rows)]
            return (
                pltpu.make_async_copy(src, buf.at[s], in_sem.at[s]),
                pltpu.make_async_copy(buf.at[s], dst, out_sem.at[s]),
            )

        for t in range(min(_NB, per_w)):
            refs(t, t % _NB)[0].start()
        waited = [False] * per_w
        for t in range(per_w):
            s = t % _NB
            cin, cout = refs(t, s)
            cin.wait()
            cout.start()
            m = t + _NB - _LAG
            if _NB <= m < per_w:
                refs(m - _NB, m % _NB)[1].wait()
                waited[m - _NB] = True
                refs(m, m % _NB)[0].start()
        for t in range(per_w):
            if not waited[t]:
                refs(t, t % _NB)[1].wait()

    out = sc_copy(x4)
    return jnp.reshape(out, (B, F * C * H, W))
